# baseline (speedup 1.0000x reference)
import functools

import jax
import jax.numpy as jnp
from jax import lax
from jax.experimental import pallas as pl
from jax.experimental.pallas import tpu as pltpu
from jax.experimental.pallas import tpu_sc as plsc

SC_NC = 2
SC_NS = 16
SC_NW = SC_NC * SC_NS

EPS = 1e-8
NEG = float(jnp.finfo(jnp.float32).min)
BIG = 1e30


def _codes(X, rotations):
    Xn = X / (jnp.linalg.norm(X, axis=-1, keepdims=True) + 1e-8)
    bits = jnp.einsum('bhld,hnd->bhln', Xn, rotations) > 0
    powers = 2 ** jnp.arange(rotations.shape[1], dtype=jnp.int32)
    return jnp.sum(bits.astype(jnp.int32) * powers, axis=-1)


def _make_sc_gather(total_rows, d):
    per_w = total_rows // SC_NW
    ch = 128
    n_ch = per_w // ch
    mesh = plsc.VectorSubcoreMesh(core_axis_name="c", subcore_axis_name="s")

    import functools as _ft
    @_ft.partial(
        pl.kernel, mesh=mesh,
        out_type=[jax.ShapeDtypeStruct((total_rows, d), jnp.float32)] * 3,
        scratch_types=[
            pltpu.VMEM((ch,), jnp.int32),
            pltpu.VMEM((ch, d), jnp.float32),
            pltpu.SemaphoreType.DMA,
            pltpu.VMEM((ch,), jnp.int32),
            pltpu.VMEM((ch, d), jnp.float32),
            pltpu.SemaphoreType.DMA,
        ],
    )
    def sc_gather(q_hbm, k_hbm, v_hbm, gq_hbm, gk_hbm,
                  qs_hbm, ks_hbm, vs_hbm,
                  idx0, rows0, sem0, idx1, rows1, sem1):
        wid = lax.axis_index("s") * SC_NC + lax.axis_index("c")
        base = wid * per_w
        bufs = ((idx0, rows0, sem0), (idx1, rows1, sem1))
        tasks = []
        for table, gidx, dst in ((q_hbm, gq_hbm, qs_hbm),
                                 (k_hbm, gk_hbm, ks_hbm),
                                 (v_hbm, gk_hbm, vs_hbm)):
            for c in range(n_ch):
                tasks.append((table, gidx, dst, c * ch))

        def fire(i):
            table, gidx, _, off = tasks[i]
            idx_v, rows_v, sem = bufs[i % 2]
            pltpu.sync_copy(gidx.at[pl.ds(base + off, ch)], idx_v)
            return pltpu.async_copy(table.at[idx_v], rows_v, sem)

        pending = fire(0)
        for i in range(len(tasks)):
            nxt = fire(i + 1) if i + 1 < len(tasks) else None
            pending.wait()
            _, _, dst, off = tasks[i]
            pltpu.sync_copy(bufs[i % 2][1], dst.at[pl.ds(base + off, ch)])
            pending = nxt

    return sc_gather


def _make_sc_scatter(total_rows, d):
    per_w = total_rows // SC_NW
    ch = 128
    n_ch = per_w // ch
    mesh = plsc.VectorSubcoreMesh(core_axis_name="c", subcore_axis_name="s")

    import functools as _ft
    @_ft.partial(
        pl.kernel, mesh=mesh,
        out_type=jax.ShapeDtypeStruct((total_rows, d), jnp.float32),
        scratch_types=[
            pltpu.VMEM((ch,), jnp.int32),
            pltpu.VMEM((ch, d), jnp.float32),
            pltpu.SemaphoreType.DMA,
            pltpu.VMEM((ch,), jnp.int32),
            pltpu.VMEM((ch, d), jnp.float32),
            pltpu.SemaphoreType.DMA,
        ],
    )
    def sc_scatter(rows_hbm, gidx_hbm, out_hbm,
                   idx0, rows0, sem0, idx1, rows1, sem1):
        wid = lax.axis_index("s") * SC_NC + lax.axis_index("c")
        base = wid * per_w
        bufs = ((idx0, rows0, sem0), (idx1, rows1, sem1))

        def fire(c):
            idx_v, rows_v, sem = bufs[c % 2]
            off = base + c * ch
            pltpu.sync_copy(gidx_hbm.at[pl.ds(off, ch)], idx_v)
            pltpu.sync_copy(rows_hbm.at[pl.ds(off, ch)], rows_v)
            return pltpu.async_copy(rows_v, out_hbm.at[idx_v], sem)

        pending = fire(0)
        for c in range(n_ch):
            nxt = fire(c + 1) if c + 1 < n_ch else None
            pending.wait()
            pending = nxt

    return sc_scatter


def _attn_kernel(lo_ref, num_ref, q_ref, qoh_ref, k_ref, koh_ref, v_ref,
                 o_ref, *, scale, n_hashes, tk):
    h = pl.program_id(0)
    i = pl.program_id(1)
    lo = lo_ref[h, i]
    num = num_ref[h, i]

    qs = q_ref[0] * scale
    qoh = qoh_ref[0]

    tq, d = qs.shape
    m0 = jnp.full((tq, 1), NEG, jnp.float32)
    l0 = jnp.zeros((tq, 1), jnp.float32)
    a0 = jnp.zeros((tq, d), jnp.float32)

    def body(t, carry):
        m, l, acc = carry
        off = (lo + t) * tk
        k = k_ref[0, pl.ds(off, tk), :]
        koh = koh_ref[0, pl.ds(off, tk), :]
        v = v_ref[0, pl.ds(off, tk), :]
        match = jnp.dot(qoh, koh.T, preferred_element_type=jnp.float32)
        s = (jnp.dot(qs, k.T, preferred_element_type=jnp.float32)
             + (match - 1.0) * 30000.0)
        m_new = jnp.maximum(m, jnp.max(s, axis=-1, keepdims=True))
        p = jnp.exp(s - m_new)
        alpha = jnp.exp(m - m_new)
        l_new = l * alpha + jnp.sum(p, axis=-1, keepdims=True)
        acc_new = acc * alpha + jnp.dot(p, v, preferred_element_type=jnp.float32)
        return m_new, l_new, acc_new

    m, l, acc = jax.lax.fori_loop(0, num, body, (m0, l0, a0))
    o_ref[0] = jnp.where(m > -15000.0, acc / ((l + EPS) * n_hashes), 0.0)


def kernel(Q, K, V, rotations):
    B, H, S, D = Q.shape
    NH = rotations.shape[1]
    NB = 2 ** NH
    TQ = 256
    TK = 256
    NQ = S // TQ
    scale = 1.0 / (D ** 0.5)

    qcode = _codes(Q, rotations)
    kcode = _codes(K, rotations)

    qorder = jnp.argsort(qcode, axis=-1).astype(jnp.int32)
    korder = jnp.argsort(kcode, axis=-1).astype(jnp.int32)

    BH_ = B * H
    head_off = (jnp.arange(BH_, dtype=jnp.int32) * S).reshape(B, H, 1)
    gq = (qorder + head_off).reshape(BH_ * S)
    gk = (korder + head_off).reshape(BH_ * S)
    sc_gather = _make_sc_gather(BH_ * S, D)
    Qs2, Ks2, Vs2 = sc_gather(
        Q.reshape(BH_ * S, D), K.reshape(BH_ * S, D), V.reshape(BH_ * S, D),
        gq, gk)
    Qs = Qs2.reshape(B, H, S, D)
    Ks = Ks2.reshape(B, H, S, D)
    Vs = Vs2.reshape(B, H, S, D)

    qsc = jnp.take_along_axis(qcode, qorder, axis=-1)
    ksc = jnp.take_along_axis(kcode, korder, axis=-1)
    bucket_ids = jnp.arange(NB, dtype=jnp.int32)
    QOH = (qsc[..., None] == bucket_ids).astype(jnp.float32)
    KOH = (ksc[..., None] == bucket_ids).astype(jnp.float32)

    buckets = jnp.arange(NB + 1, dtype=jnp.int32)
    kstart = jnp.sum(kcode[..., None] < buckets, axis=-2).astype(jnp.int32)

    qst = qsc.reshape(B, H, NQ, TQ)
    b_first = qst[..., 0]
    b_last = qst[..., -1]
    lo_row = jnp.take_along_axis(kstart, b_first, axis=-1)
    hi_row = jnp.take_along_axis(kstart, b_last + 1, axis=-1)
    lo = lo_row // TK
    num = jnp.where(hi_row > lo_row, (hi_row - 1) // TK - lo + 1, 0)

    BH = B * H
    q = Qs.reshape(BH, S, D)
    k = Ks.reshape(BH, S, D)
    v = Vs.reshape(BH, S, D)
    qoh = QOH.reshape(BH, S, NB)
    koh = KOH.reshape(BH, S, NB)
    lo = lo.reshape(BH, NQ).astype(jnp.int32)
    num = num.reshape(BH, NQ).astype(jnp.int32)

    grid_spec = pltpu.PrefetchScalarGridSpec(
        num_scalar_prefetch=2,
        grid=(BH, NQ),
        in_specs=[
            pl.BlockSpec((1, TQ, D), lambda h, i, lo_r, num_r: (h, i, 0)),
            pl.BlockSpec((1, TQ, NB), lambda h, i, lo_r, num_r: (h, i, 0)),
            pl.BlockSpec((1, S, D), lambda h, i, lo_r, num_r: (h, 0, 0)),
            pl.BlockSpec((1, S, NB), lambda h, i, lo_r, num_r: (h, 0, 0)),
            pl.BlockSpec((1, S, D), lambda h, i, lo_r, num_r: (h, 0, 0)),
        ],
        out_specs=pl.BlockSpec((1, TQ, D), lambda h, i, lo_r, num_r: (h, i, 0)),
    )
    out_sorted = pl.pallas_call(
        functools.partial(_attn_kernel, scale=scale, n_hashes=NH, tk=TK),
        grid_spec=grid_spec,
        out_shape=jax.ShapeDtypeStruct((BH, S, D), jnp.float32),
    )(lo, num, q, qoh, k, koh, v)

    sc_scatter = _make_sc_scatter(BH_ * S, D)
    out = sc_scatter(out_sorted.reshape(BH_ * S, D), gq)
    return out.reshape(B, H, S, D)
